# baseline (speedup 1.0000x reference)
import functools

import jax
import jax.numpy as jnp
from jax.experimental import pallas as pl
from jax.experimental.pallas import tpu as pltpu

_TGT_BLOCK = 2048


def _dist_loss_kernel(st_ref, t_ref, out_ref):
    jb = pl.program_id(0)

    st = st_ref[...]
    ns = jnp.sum(st * st, axis=0, keepdims=True)
    b = jnp.concatenate([st, ns], axis=0)

    t = t_ref[...]
    ones = jnp.ones((t.shape[0], 1), dtype=t.dtype)
    a = jnp.concatenate([-2.0 * t, ones], axis=1)

    a_hi = a.astype(jnp.bfloat16)
    a_lo = (a - a_hi.astype(jnp.float32)).astype(jnp.bfloat16)
    b_hi = b.astype(jnp.bfloat16)
    b_lo = (b - b_hi.astype(jnp.float32)).astype(jnp.bfloat16)
    aa = jnp.concatenate([a_hi, a_hi, a_lo, a_lo], axis=1)
    bb = jnp.concatenate([b_hi, b_lo, b_hi, b_lo], axis=0)

    n = bb.shape[1]
    csz = 1024
    mins = []
    for c in range(n // csz):
        mc = jax.lax.dot_general(
            aa, bb[:, c * csz:(c + 1) * csz], (((1,), (0,)), ((), ())),
            preferred_element_type=jnp.float32)
        mins.append(jnp.min(mc, axis=1, keepdims=True))
    colmin = jax.tree.reduce(jnp.minimum, mins)
    nt = jnp.sum(t * t, axis=1, keepdims=True)
    partial = jnp.sum(colmin + nt, axis=0, keepdims=True)

    @pl.when(jb == 0)
    def _init():
        out_ref[...] = jnp.zeros_like(out_ref)

    out_ref[...] += partial


@functools.partial(jax.jit, static_argnames=())
def kernel(surfaces, targets):
    s_flat = surfaces.reshape(-1, 3)
    st = s_flat.T
    m = targets.shape[0]
    n = s_flat.shape[0]
    grid = (m // _TGT_BLOCK,)

    out = pl.pallas_call(
        _dist_loss_kernel,
        grid=grid,
        in_specs=[
            pl.BlockSpec((3, n), lambda j: (0, 0)),
            pl.BlockSpec((_TGT_BLOCK, 3), lambda j: (j, 0)),
        ],
        out_specs=pl.BlockSpec((1, 1), lambda j: (0, 0)),
        out_shape=jax.ShapeDtypeStruct((1, 1), jnp.float32),
    )(st, targets)
    return out[0, 0]


# speedup vs baseline: 1.0129x; 1.0129x over previous
import functools

import jax
import jax.numpy as jnp
from jax.experimental import pallas as pl
from jax.experimental.pallas import tpu as pltpu

_TGT_BLOCK = 4096


def _dist_loss_kernel(st_ref, t_ref, out_ref):
    jb = pl.program_id(0)

    st = st_ref[...]
    ns = jnp.sum(st * st, axis=0, keepdims=True)
    b = jnp.concatenate([st, ns], axis=0)

    t = t_ref[...]
    ones = jnp.ones((t.shape[0], 1), dtype=t.dtype)
    a = jnp.concatenate([-2.0 * t, ones], axis=1)

    a_hi = a.astype(jnp.bfloat16)
    a_lo = (a - a_hi.astype(jnp.float32)).astype(jnp.bfloat16)
    b_hi = b.astype(jnp.bfloat16)
    b_lo = (b - b_hi.astype(jnp.float32)).astype(jnp.bfloat16)
    aa = jnp.concatenate([a_hi, a_hi, a_lo, a_lo], axis=1)
    bb = jnp.concatenate([b_hi, b_lo, b_hi, b_lo], axis=0)

    m = jax.lax.dot_general(
        aa, bb, (((1,), (0,)), ((), ())),
        preferred_element_type=jnp.float32)
    colmin = jnp.min(m, axis=1, keepdims=True)
    nt = jnp.sum(t * t, axis=1, keepdims=True)
    partial = jnp.sum(colmin + nt, axis=0, keepdims=True)

    @pl.when(jb == 0)
    def _init():
        out_ref[...] = jnp.zeros_like(out_ref)

    out_ref[...] += partial


@functools.partial(jax.jit, static_argnames=())
def kernel(surfaces, targets):
    s_flat = surfaces.reshape(-1, 3)
    st = s_flat.T
    m = targets.shape[0]
    n = s_flat.shape[0]
    grid = (m // _TGT_BLOCK,)

    out = pl.pallas_call(
        _dist_loss_kernel,
        grid=grid,
        in_specs=[
            pl.BlockSpec((3, n), lambda j: (0, 0)),
            pl.BlockSpec((_TGT_BLOCK, 3), lambda j: (j, 0)),
        ],
        out_specs=pl.BlockSpec((1, 1), lambda j: (0, 0)),
        out_shape=jax.ShapeDtypeStruct((1, 1), jnp.float32),
    )(st, targets)
    return out[0, 0]


# speedup vs baseline: 1.0151x; 1.0021x over previous
import functools

import jax
import jax.numpy as jnp
from jax.experimental import pallas as pl

_TGT_BLOCK = 4096


def _dist_loss_kernel(st_ref, t_ref, out_ref):
    jb = pl.program_id(0)

    st = st_ref[...]
    ns = jnp.sum(st * st, axis=0, keepdims=True)
    b = jnp.concatenate([st, ns], axis=0)

    t = t_ref[...]
    ones = jnp.ones((t.shape[0], 1), dtype=t.dtype)
    a = jnp.concatenate([-2.0 * t, ones], axis=1)

    a_hi = a.astype(jnp.bfloat16)
    a_lo = (a - a_hi.astype(jnp.float32)).astype(jnp.bfloat16)
    b_hi = b.astype(jnp.bfloat16)
    b_lo = (b - b_hi.astype(jnp.float32)).astype(jnp.bfloat16)
    aa = jnp.concatenate([a_hi, a_hi, a_lo, a_lo], axis=1)
    bb = jnp.concatenate([b_hi, b_lo, b_hi, b_lo], axis=0)

    m = jax.lax.dot_general(
        aa, bb, (((1,), (0,)), ((), ())),
        preferred_element_type=jnp.float32)
    colmin = jnp.min(m, axis=1, keepdims=True)
    nt = jnp.sum(t * t, axis=1, keepdims=True)
    partial = jnp.sum(colmin + nt, axis=0, keepdims=True)

    @pl.when(jb == 0)
    def _init():
        out_ref[...] = jnp.zeros_like(out_ref)

    out_ref[...] += partial


@functools.partial(jax.jit, static_argnames=())
def kernel(surfaces, targets):
    s_flat = surfaces.reshape(-1, 3)
    st = s_flat.T
    m = targets.shape[0]
    n = s_flat.shape[0]
    grid = (m // _TGT_BLOCK,)

    out = pl.pallas_call(
        _dist_loss_kernel,
        grid=grid,
        in_specs=[
            pl.BlockSpec((3, n), lambda j: (0, 0)),
            pl.BlockSpec((_TGT_BLOCK, 3), lambda j: (j, 0)),
        ],
        out_specs=pl.BlockSpec((1, 1), lambda j: (0, 0)),
        out_shape=jax.ShapeDtypeStruct((1, 1), jnp.float32),
    )(st, targets)
    return out[0, 0]
